# baseline (speedup 1.0000x reference)
import functools

import jax
import jax.numpy as jnp
from jax import lax
from jax.experimental import pallas as pl
from jax.experimental.pallas import tpu as pltpu
from jax.experimental.pallas import tpu_sc as plsc

DEPTH = 32
NUM_TOKENS = 4096 * 200
NC = 2
NS = 16
NW = NC * NS
PER_W = NUM_TOKENS // NW
NBUF = 8
SUB = 200
NCHUNK = PER_W // SUB
NROUND = NCHUNK // NBUF

_mesh = plsc.VectorSubcoreMesh(core_axis_name="c", subcore_axis_name="s")


@functools.partial(
    pl.kernel,
    mesh=_mesh,
    compiler_params=pltpu.CompilerParams(use_tc_tiling_on_sc=False),
    out_type=jax.ShapeDtypeStruct((NUM_TOKENS, DEPTH), jnp.float32),
    scratch_types=[
        pltpu.VMEM((NCHUNK, SUB), jnp.int32),
        pltpu.VMEM((NBUF, SUB, DEPTH), jnp.float32),
        pltpu.SemaphoreType.DMA((NBUF,)),
        pltpu.SemaphoreType.DMA((NBUF,)),
    ],
)
def _embed_lookup(idx_hbm, table_hbm, out_hbm, idx_v, rows_v, sg, so):
    wid = lax.axis_index("s") * NC + lax.axis_index("c")
    base = wid * PER_W
    pltpu.sync_copy(idx_hbm.at[wid], idx_v)

    def g_copy(i, b):
        return pltpu.make_async_copy(
            table_hbm.at[idx_v.at[i]], rows_v.at[b], sg.at[b])

    def o_copy(i, b):
        return pltpu.make_async_copy(
            rows_v.at[b], out_hbm.at[pl.ds(base + i * SUB, SUB)], so.at[b])

    for b in range(NBUF):
        g_copy(b, b).start()

    def body(k, carry):
        i0 = k * NBUF
        for b in range(NBUF):
            i = i0 + b
            g_copy(i, b).wait()
            o_copy(i, b).start()
            o_copy(i, b).wait()
            g_copy(i + NBUF, b).start()
        return carry

    lax.fori_loop(0, NROUND - 1, body, 0)

    i0 = (NROUND - 1) * NBUF
    for b in range(NBUF):
        i = i0 + b
        g_copy(i, b).wait()
        o_copy(i, b).start()
    for b in range(NBUF):
        o_copy(i0 + b, b).wait()


def kernel(texts_tokenized, table):
    idx = texts_tokenized.reshape(NW, NCHUNK, SUB).astype(jnp.int32)
    out = _embed_lookup(idx, table)
    return out.reshape(texts_tokenized.shape + (DEPTH,))
